# baseline (speedup 1.0000x reference)
import jax
import jax.numpy as jnp
from jax import lax
from jax.experimental import pallas as pl
from jax.experimental.pallas import tpu as pltpu
from jax.experimental.pallas import tpu_sc as plsc

N_ENTITIES = 1000000
N_RELATIONS = 1000
D = 64
B = 16384
N_NEG = 4
NB = B * N_NEG

NC = 2
NS = 16
NW = NC * NS
L = 16

P = B // NW
C = 64
NSTAGE = (P + N_NEG * P) // C
NPOS = P // C
NROW = (1 + N_NEG) * P // 128
NRS = P // 128
RING = 3


def _neg_sqrt(x):
    i = plsc.bitcast(x, jnp.int32)
    i = 0x5F3759DF - lax.shift_right_arithmetic(i, 1)
    y = plsc.bitcast(i, jnp.float32)
    half = x * (-0.5)
    for _ in range(3):
        y = y * (1.5 + half * y * y)
    return -(x * y)


NCHUNK = 7812
NE2 = N_ENTITIES // 2


def _tr_kernel(entT, tail32, ent2, tbuf, pbuf, tailbuf, sem_r0, sem_r1, sem_r2, sem_w):
    sems_r = [sem_r0, sem_r1, sem_r2]
    wid = lax.axis_index("s") * NC + lax.axis_index("c")
    iota = lax.iota(jnp.int32, L)
    nd = NCHUNK // 2
    my_n = jnp.where(wid < nd - (nd // NW) * NW, nd // NW + 1, nd // NW)

    rots = [(iota + r) & (L - 1) for r in range(L)]

    def fire_read(i):
        chunk = i * NW + wid
        for q in range(RING):
            @pl.when(i % RING == q)
            def _(q=q):
                pltpu.async_copy(
                    entT.at[pl.ds(0, D), pl.ds(chunk * 256, 256)],
                    tbuf.at[q],
                    sems_r[q],
                )

    def wait_read(i):
        chunk = i * NW + wid
        for q in range(RING):
            @pl.when(i % RING == q)
            def _(q=q):
                pltpu.make_async_copy(
                    entT.at[pl.ds(0, D), pl.ds(chunk * 256, 256)],
                    tbuf.at[q],
                    sems_r[q],
                ).wait()

    def fire_write(i):
        sp = i % RING
        chunk = i * NW + wid
        pltpu.async_copy(pbuf.at[sp], ent2.at[pl.ds(chunk * 128, 128)], sem_w)

    def wait_write(i):
        sp = i % RING
        chunk = i * NW + wid
        pltpu.make_async_copy(
            pbuf.at[sp], ent2.at[pl.ds(chunk * 128, 128)], sem_w
        ).wait()

    fire_read(jnp.int32(0))
    fire_read(jnp.int32(1))

    def body(i, _):
        @pl.when(i + 2 < my_n)
        def _():
            fire_read(i + 2)

        wait_read(i)

        @pl.when(i >= 2)
        def _():
            wait_write(i - 2)

        sp16 = jnp.full((L,), i % RING, jnp.int32)
        for fb in range(D // L):
            f16 = fb * L + iota

            def eb_body(eb, _):
                vs = [
                    plsc.load_gather(tbuf, [sp16, f16, eb * L + rots[r]])
                    for r in range(L)
                ]
                for r in range(L):
                    rot = rots[r]
                    dst_r = eb * (L // 2) + lax.shift_right_logical(rot, 1)
                    dst_c = lax.shift_left(rot & 1, 6) + f16
                    plsc.store_scatter(pbuf, [sp16, dst_r, dst_c], vs[r])
                return ()

            lax.fori_loop(0, 256 // L, eb_body, (), unroll=False)
        fire_write(i)
        return ()

    lax.fori_loop(0, my_n, body, (), unroll=False)
    wait_write(my_n - 2)
    wait_write(my_n - 1)

    @pl.when(wid == 0)
    def _():
        pltpu.sync_copy(tail32, tailbuf)
        pltpu.sync_copy(tailbuf, ent2.at[pl.ds(NCHUNK * D, 32)])


def _sc_kernel(
    heads,
    tails,
    relations,
    negative_head,
    negative_tails,
    entity2,
    relation2,
    pos_out,
    neg_out,
    idx_h,
    idx_t,
    idx_r,
    idx2_h,
    idx2_t,
    idx2_r,
    hbuf,
    tbuf,
    rbuf,
    scores_v,
    sem_i,
    sem_r,
    sem_g0,
    sem_g1,
    sem_g2,
    sem_o,
):
    sems_g = [sem_g0, sem_g1, sem_g2]
    wid = lax.axis_index("s") * NC + lax.axis_index("c")
    base = wid * P
    iota = lax.iota(jnp.int32, L)

    idx_copies = []
    for j in range(NRS):
        idx_copies.append(
            pltpu.async_copy(
                relations.at[pl.ds(base + j * 128, 128)], idx_r.at[j], sem_i
            )
        )
        idx_copies.append(
            pltpu.async_copy(heads.at[pl.ds(base + j * 128, 128)], idx_h.at[j], sem_i)
        )
        idx_copies.append(
            pltpu.async_copy(tails.at[pl.ds(base + j * 128, 128)], idx_t.at[j], sem_i)
        )
    for k in range(N_NEG):
        for j in range(NRS):
            src = k * B + base + j * 128
            row = NRS + k * NRS + j
            idx_copies.append(
                pltpu.async_copy(
                    negative_head.at[pl.ds(src, 128)], idx_h.at[row], sem_i
                )
            )
            idx_copies.append(
                pltpu.async_copy(
                    negative_tails.at[pl.ds(src, 128)], idx_t.at[row], sem_i
                )
            )
    for c in idx_copies:
        c.wait()

    for j in range(NRS):
        for v in range(128 // L):
            sl = pl.ds(v * L, L)
            idx2_r[j, sl] = lax.shift_right_logical(idx_r[j, sl], 1)
    r_copies = [
        pltpu.async_copy(
            relation2.at[idx2_r.at[j]], rbuf.at[pl.ds(j * 128, 128)], sem_r
        )
        for j in range(NRS)
    ]
    for c in r_copies:
        c.wait()

    def fire(s):
        sp = s % RING
        srow = s // 2
        hh = (s % 2) * C
        for v in range(C // L):
            idx2_h[sp, pl.ds(v * L, L)] = lax.shift_right_logical(
                idx_h[srow, pl.ds(hh + v * L, L)], 1
            )
            idx2_t[sp, pl.ds(v * L, L)] = lax.shift_right_logical(
                idx_t[srow, pl.ds(hh + v * L, L)], 1
            )
        for q in range(RING):
            @pl.when(sp == q)
            def _(q=q):
                pltpu.async_copy(
                    entity2.at[idx2_h.at[q]], hbuf.at[pl.ds(q * C, C)], sems_g[q]
                )
                pltpu.async_copy(
                    entity2.at[idx2_t.at[q]], tbuf.at[pl.ds(q * C, C)], sems_g[q]
                )

    def wait_stage(s):
        sp = s % RING
        for q in range(RING):
            @pl.when(sp == q)
            def _(q=q):
                pltpu.make_async_copy(
                    entity2.at[idx2_h.at[q]], hbuf.at[pl.ds(q * C, C)], sems_g[q]
                ).wait()
                pltpu.make_async_copy(
                    entity2.at[idx2_t.at[q]], tbuf.at[pl.ds(q * C, C)], sems_g[q]
                ).wait()

    fire(jnp.int32(0))
    fire(jnp.int32(1))

    def stage_body(s, _):
        wait_stage(s)

        @pl.when(s < NSTAGE - 2)
        def _():
            fire(s + 2)

        sp = s % 2
        srow = s // 2
        hh = (s % 2) * C
        soff16 = jnp.full((L,), (s % RING) * C, jnp.int32)
        rel0 = (s % NPOS) * C
        roff16 = jnp.full((L,), rel0, jnp.int32)
        rrow = (s % NPOS) // 2
        rh = ((s % NPOS) % 2) * C

        def group(g, _):
            rows = g * L + iota
            srows = soff16 + rows
            prows = roff16 + rows
            ph = (idx_h[srow, pl.ds(hh + g * L, L)] & 1) * D
            pt = (idx_t[srow, pl.ds(hh + g * L, L)] & 1) * D
            pr = (idx_r[rrow, pl.ds(rh + g * L, L)] & 1) * D
            acc = jnp.zeros((L,), jnp.float32)
            for d in range(D):
                cols = (iota + d) & (D - 1)
                hv = plsc.load_gather(hbuf, [srows, ph + cols])
                tv = plsc.load_gather(tbuf, [srows, pt + cols])
                rv = plsc.load_gather(rbuf, [prows, pr + cols])
                diff = hv + rv - tv
                acc = acc + diff * diff
            scores_v[s, pl.ds(g * L, L)] = _neg_sqrt(acc + 1e-12)
            return ()

        lax.fori_loop(0, C // L, group, (), unroll=False)

        @pl.when(s < NPOS)
        def _():
            for k in range(N_NEG):
                pltpu.async_copy(
                    scores_v.at[s], pos_out.at[pl.ds(k * B + base + s * C, C)], sem_o
                )

        @pl.when(s >= NPOS)
        def _():
            k2 = s - NPOS
            dst = base + (k2 // NPOS) * B + (k2 % NPOS) * C
            pltpu.async_copy(scores_v.at[s], neg_out.at[pl.ds(dst, C)], sem_o)

        return ()

    lax.fori_loop(0, NSTAGE, stage_body, (), unroll=False)

    for s in range(NPOS):
        for k in range(N_NEG):
            pltpu.make_async_copy(
                scores_v.at[s], pos_out.at[pl.ds(k * B + base + s * C, C)], sem_o
            ).wait()
    for s in range(NPOS, NSTAGE):
        k2 = s - NPOS
        dst = base + (k2 // NPOS) * B + (k2 % NPOS) * C
        pltpu.make_async_copy(
            scores_v.at[s], neg_out.at[pl.ds(dst, C)], sem_o
        ).wait()


@jax.jit
def _run(heads, tails, relations, negative_head, negative_tails, entT, tail32, relation2):
    mesh = plsc.VectorSubcoreMesh(
        core_axis_name="c", subcore_axis_name="s", num_cores=NC, num_subcores=NS
    )
    tr = pl.kernel(
        _tr_kernel,
        out_type=jax.ShapeDtypeStruct((NE2 + 32, 2 * D), jnp.float32),
        mesh=mesh,
        compiler_params=pltpu.CompilerParams(
            needs_layout_passes=False, use_tc_tiling_on_sc=True
        ),
        scratch_types=[
            pltpu.VMEM((RING, D, 256), jnp.float32),
            pltpu.VMEM((RING, 128, 128), jnp.float32),
            pltpu.VMEM((32, 128), jnp.float32),
            pltpu.SemaphoreType.DMA,
            pltpu.SemaphoreType.DMA,
            pltpu.SemaphoreType.DMA,
            pltpu.SemaphoreType.DMA,
        ],
    )
    ent2 = tr(entT, tail32)
    f = pl.kernel(
        _sc_kernel,
        out_type=(
            jax.ShapeDtypeStruct((NB,), jnp.float32),
            jax.ShapeDtypeStruct((NB,), jnp.float32),
        ),
        mesh=mesh,
        compiler_params=pltpu.CompilerParams(
            needs_layout_passes=False, use_tc_tiling_on_sc=True
        ),
        scratch_types=[
            pltpu.VMEM((NROW, 128), jnp.int32),
            pltpu.VMEM((NROW, 128), jnp.int32),
            pltpu.VMEM((NRS, 128), jnp.int32),
            pltpu.VMEM((RING, C), jnp.int32),
            pltpu.VMEM((RING, C), jnp.int32),
            pltpu.VMEM((NRS, 128), jnp.int32),
            pltpu.VMEM((RING * C, 2 * D), jnp.float32),
            pltpu.VMEM((RING * C, 2 * D), jnp.float32),
            pltpu.VMEM((P, 2 * D), jnp.float32),
            pltpu.VMEM((NSTAGE, C), jnp.float32),
            pltpu.SemaphoreType.DMA,
            pltpu.SemaphoreType.DMA,
            pltpu.SemaphoreType.DMA,
            pltpu.SemaphoreType.DMA,
            pltpu.SemaphoreType.DMA,
            pltpu.SemaphoreType.DMA,
        ],
    )
    return f(heads, tails, relations, negative_head, negative_tails, ent2, relation2)


def kernel(heads, tails, relations, negative_head, negative_tails, entity_emb, relation_emb):
    return _run(
        heads.astype(jnp.int32),
        tails.astype(jnp.int32),
        relations.astype(jnp.int32),
        negative_head.astype(jnp.int32),
        negative_tails.astype(jnp.int32),
        entity_emb.T,
        entity_emb[NCHUNK * 128 :].reshape(32, 2 * D),
        relation_emb.reshape(N_RELATIONS // 2, 2 * D),
    )


# speedup vs baseline: 1.1524x; 1.1524x over previous
import jax
import jax.numpy as jnp
from jax import lax
from jax.experimental import pallas as pl
from jax.experimental.pallas import tpu as pltpu
from jax.experimental.pallas import tpu_sc as plsc

N_ENTITIES = 1000000
N_RELATIONS = 1000
D = 64
B = 16384
N_NEG = 4
NB = B * N_NEG

NC = 2
NS = 16
NW = NC * NS
L = 16

P = B // NW
C = 64
NSTAGE = (P + N_NEG * P) // C
NPOS = P // C
NROW = (1 + N_NEG) * P // 128
NRS = P // 128
RING = 3


def _neg_sqrt(x):
    i = plsc.bitcast(x, jnp.int32)
    i = 0x5F3759DF - lax.shift_right_arithmetic(i, 1)
    y = plsc.bitcast(i, jnp.float32)
    half = x * (-0.5)
    for _ in range(3):
        y = y * (1.5 + half * y * y)
    return -(x * y)


NCHUNK = 7812
NE2 = N_ENTITIES // 2


def _tr_kernel(entT, tail32, ent2, tbuf, pbuf, tailbuf, sem_r, sem_w):
    wid = lax.axis_index("s") * NC + lax.axis_index("c")
    iota = lax.iota(jnp.int32, L)
    nd = NCHUNK // 2
    my_n = jnp.where(wid < nd - (nd // NW) * NW, nd // NW + 1, nd // NW)

    rots = [(iota + r) & (L - 1) for r in range(L)]

    def fire_read(i):
        sp = i % 2
        chunk = i * NW + wid
        pltpu.async_copy(
            entT.at[pl.ds(0, D), pl.ds(chunk * 256, 256)], tbuf.at[sp], sem_r
        )

    def wait_read(i):
        sp = i % 2
        chunk = i * NW + wid
        pltpu.make_async_copy(
            entT.at[pl.ds(0, D), pl.ds(chunk * 256, 256)], tbuf.at[sp], sem_r
        ).wait()

    def fire_write(i):
        sp = i % 2
        chunk = i * NW + wid
        pltpu.async_copy(pbuf.at[sp], ent2.at[pl.ds(chunk * 128, 128)], sem_w)

    def wait_write(i):
        sp = i % 2
        chunk = i * NW + wid
        pltpu.make_async_copy(
            pbuf.at[sp], ent2.at[pl.ds(chunk * 128, 128)], sem_w
        ).wait()

    fire_read(jnp.int32(0))

    def body(i, _):
        @pl.when(i + 1 < my_n)
        def _():
            fire_read(i + 1)

        wait_read(i)

        @pl.when(i >= 2)
        def _():
            wait_write(i - 2)

        sp16 = jnp.full((L,), i % 2, jnp.int32)
        for fb in range(D // L):
            f16 = fb * L + iota

            def eb_body(eb, _):
                vs = [
                    plsc.load_gather(tbuf, [sp16, f16, eb * L + rots[r]])
                    for r in range(L)
                ]
                for r in range(L):
                    rot = rots[r]
                    dst_r = eb * (L // 2) + lax.shift_right_logical(rot, 1)
                    dst_c = lax.shift_left(rot & 1, 6) + f16
                    plsc.store_scatter(pbuf, [sp16, dst_r, dst_c], vs[r])
                return ()

            lax.fori_loop(0, 256 // L, eb_body, (), unroll=False)
        fire_write(i)
        return ()

    lax.fori_loop(0, my_n, body, (), unroll=False)
    wait_write(my_n - 2)
    wait_write(my_n - 1)

    @pl.when(wid == 0)
    def _():
        pltpu.sync_copy(tail32, tailbuf)
        pltpu.sync_copy(tailbuf, ent2.at[pl.ds(NCHUNK * D, 32)])


def _sc_kernel(
    heads,
    tails,
    relations,
    negative_head,
    negative_tails,
    entity2,
    relation2,
    pos_out,
    neg_out,
    idx_h,
    idx_t,
    idx_r,
    idx2_h,
    idx2_t,
    idx2_r,
    hbuf,
    tbuf,
    rbuf,
    scores_v,
    sem_i,
    sem_r,
    sem_g0,
    sem_g1,
    sem_g2,
    sem_o,
):
    sems_g = [sem_g0, sem_g1, sem_g2]
    wid = lax.axis_index("s") * NC + lax.axis_index("c")
    base = wid * P
    iota = lax.iota(jnp.int32, L)

    idx_copies = []
    for j in range(NRS):
        idx_copies.append(
            pltpu.async_copy(
                relations.at[pl.ds(base + j * 128, 128)], idx_r.at[j], sem_i
            )
        )
        idx_copies.append(
            pltpu.async_copy(heads.at[pl.ds(base + j * 128, 128)], idx_h.at[j], sem_i)
        )
        idx_copies.append(
            pltpu.async_copy(tails.at[pl.ds(base + j * 128, 128)], idx_t.at[j], sem_i)
        )
    for k in range(N_NEG):
        for j in range(NRS):
            src = k * B + base + j * 128
            row = NRS + k * NRS + j
            idx_copies.append(
                pltpu.async_copy(
                    negative_head.at[pl.ds(src, 128)], idx_h.at[row], sem_i
                )
            )
            idx_copies.append(
                pltpu.async_copy(
                    negative_tails.at[pl.ds(src, 128)], idx_t.at[row], sem_i
                )
            )
    for c in idx_copies:
        c.wait()

    for j in range(NRS):
        for v in range(128 // L):
            sl = pl.ds(v * L, L)
            idx2_r[j, sl] = lax.shift_right_logical(idx_r[j, sl], 1)
    r_copies = [
        pltpu.async_copy(
            relation2.at[idx2_r.at[j]], rbuf.at[pl.ds(j * 128, 128)], sem_r
        )
        for j in range(NRS)
    ]
    for c in r_copies:
        c.wait()

    def fire(s):
        sp = s % RING
        srow = s // 2
        hh = (s % 2) * C
        for v in range(C // L):
            idx2_h[sp, pl.ds(v * L, L)] = lax.shift_right_logical(
                idx_h[srow, pl.ds(hh + v * L, L)], 1
            )
            idx2_t[sp, pl.ds(v * L, L)] = lax.shift_right_logical(
                idx_t[srow, pl.ds(hh + v * L, L)], 1
            )
        for q in range(RING):
            @pl.when(sp == q)
            def _(q=q):
                pltpu.async_copy(
                    entity2.at[idx2_h.at[q]], hbuf.at[pl.ds(q * C, C)], sems_g[q]
                )
                pltpu.async_copy(
                    entity2.at[idx2_t.at[q]], tbuf.at[pl.ds(q * C, C)], sems_g[q]
                )

    def wait_stage(s):
        sp = s % RING
        for q in range(RING):
            @pl.when(sp == q)
            def _(q=q):
                pltpu.make_async_copy(
                    entity2.at[idx2_h.at[q]], hbuf.at[pl.ds(q * C, C)], sems_g[q]
                ).wait()
                pltpu.make_async_copy(
                    entity2.at[idx2_t.at[q]], tbuf.at[pl.ds(q * C, C)], sems_g[q]
                ).wait()

    fire(jnp.int32(0))
    fire(jnp.int32(1))

    def stage_body(s, _):
        wait_stage(s)

        @pl.when(s < NSTAGE - 2)
        def _():
            fire(s + 2)

        sp = s % 2
        srow = s // 2
        hh = (s % 2) * C
        soff16 = jnp.full((L,), (s % RING) * C, jnp.int32)
        rel0 = (s % NPOS) * C
        roff16 = jnp.full((L,), rel0, jnp.int32)
        rrow = (s % NPOS) // 2
        rh = ((s % NPOS) % 2) * C

        def group(g, _):
            rows = g * L + iota
            srows = soff16 + rows
            prows = roff16 + rows
            ph = (idx_h[srow, pl.ds(hh + g * L, L)] & 1) * D
            pt = (idx_t[srow, pl.ds(hh + g * L, L)] & 1) * D
            pr = (idx_r[rrow, pl.ds(rh + g * L, L)] & 1) * D
            acc = jnp.zeros((L,), jnp.float32)
            for d in range(D):
                cols = (iota + d) & (D - 1)
                hv = plsc.load_gather(hbuf, [srows, ph + cols])
                tv = plsc.load_gather(tbuf, [srows, pt + cols])
                rv = plsc.load_gather(rbuf, [prows, pr + cols])
                diff = hv + rv - tv
                acc = acc + diff * diff
            scores_v[s, pl.ds(g * L, L)] = _neg_sqrt(acc + 1e-12)
            return ()

        lax.fori_loop(0, C // L, group, (), unroll=False)

        @pl.when(s < NPOS)
        def _():
            for k in range(N_NEG):
                pltpu.async_copy(
                    scores_v.at[s], pos_out.at[pl.ds(k * B + base + s * C, C)], sem_o
                )

        @pl.when(s >= NPOS)
        def _():
            k2 = s - NPOS
            dst = base + (k2 // NPOS) * B + (k2 % NPOS) * C
            pltpu.async_copy(scores_v.at[s], neg_out.at[pl.ds(dst, C)], sem_o)

        return ()

    lax.fori_loop(0, NSTAGE, stage_body, (), unroll=False)

    for s in range(NPOS):
        for k in range(N_NEG):
            pltpu.make_async_copy(
                scores_v.at[s], pos_out.at[pl.ds(k * B + base + s * C, C)], sem_o
            ).wait()
    for s in range(NPOS, NSTAGE):
        k2 = s - NPOS
        dst = base + (k2 // NPOS) * B + (k2 % NPOS) * C
        pltpu.make_async_copy(
            scores_v.at[s], neg_out.at[pl.ds(dst, C)], sem_o
        ).wait()


@jax.jit
def _run(heads, tails, relations, negative_head, negative_tails, entT, tail32, relation2):
    mesh = plsc.VectorSubcoreMesh(
        core_axis_name="c", subcore_axis_name="s", num_cores=NC, num_subcores=NS
    )
    tr = pl.kernel(
        _tr_kernel,
        out_type=jax.ShapeDtypeStruct((NE2 + 32, 2 * D), jnp.float32),
        mesh=mesh,
        compiler_params=pltpu.CompilerParams(
            needs_layout_passes=False, use_tc_tiling_on_sc=True
        ),
        scratch_types=[
            pltpu.VMEM((2, D, 256), jnp.float32),
            pltpu.VMEM((2, 128, 128), jnp.float32),
            pltpu.VMEM((32, 128), jnp.float32),
            pltpu.SemaphoreType.DMA,
            pltpu.SemaphoreType.DMA,
        ],
    )
    ent2 = tr(entT, tail32)
    f = pl.kernel(
        _sc_kernel,
        out_type=(
            jax.ShapeDtypeStruct((NB,), jnp.float32),
            jax.ShapeDtypeStruct((NB,), jnp.float32),
        ),
        mesh=mesh,
        compiler_params=pltpu.CompilerParams(
            needs_layout_passes=False, use_tc_tiling_on_sc=True
        ),
        scratch_types=[
            pltpu.VMEM((NROW, 128), jnp.int32),
            pltpu.VMEM((NROW, 128), jnp.int32),
            pltpu.VMEM((NRS, 128), jnp.int32),
            pltpu.VMEM((RING, C), jnp.int32),
            pltpu.VMEM((RING, C), jnp.int32),
            pltpu.VMEM((NRS, 128), jnp.int32),
            pltpu.VMEM((RING * C, 2 * D), jnp.float32),
            pltpu.VMEM((RING * C, 2 * D), jnp.float32),
            pltpu.VMEM((P, 2 * D), jnp.float32),
            pltpu.VMEM((NSTAGE, C), jnp.float32),
            pltpu.SemaphoreType.DMA,
            pltpu.SemaphoreType.DMA,
            pltpu.SemaphoreType.DMA,
            pltpu.SemaphoreType.DMA,
            pltpu.SemaphoreType.DMA,
            pltpu.SemaphoreType.DMA,
        ],
    )
    return f(heads, tails, relations, negative_head, negative_tails, ent2, relation2)


def kernel(heads, tails, relations, negative_head, negative_tails, entity_emb, relation_emb):
    return _run(
        heads.astype(jnp.int32),
        tails.astype(jnp.int32),
        relations.astype(jnp.int32),
        negative_head.astype(jnp.int32),
        negative_tails.astype(jnp.int32),
        entity_emb.T,
        entity_emb[NCHUNK * 128 :].reshape(32, 2 * D),
        relation_emb.reshape(N_RELATIONS // 2, 2 * D),
    )
